# Initial kernel scaffold; baseline (speedup 1.0000x reference)
import jax
import jax.numpy as jnp
from jax.experimental import pallas as pl


def kernel(x, edge_index, W, b):
    raise NotImplementedError("write your pallas kernel here")


# speedup vs baseline: 20.9332x; 20.9332x over previous
import functools

import jax
import jax.numpy as jnp
from jax import lax
from jax.experimental import pallas as pl
from jax.experimental.pallas import tpu as pltpu
from jax.experimental.pallas import tpu_sc as plsc

NC = 2
NS = 16
NW = NC * NS
CHUNK = 128

_MESH = plsc.VectorSubcoreMesh(core_axis_name="c", subcore_axis_name="s")


def _hist_kernel(n2, nchunk, stripe):
    @functools.partial(
        pl.kernel,
        out_type=jax.ShapeDtypeStruct((NC * n2,), jnp.float32),
        mesh=_MESH,
        scratch_types=[
            pltpu.VMEM((nchunk, CHUNK), jnp.int32),
            pltpu.VMEM((CHUNK,), jnp.float32),
            pltpu.VMEM((stripe,), jnp.float32),
            pltpu.VMEM_SHARED((n2,), jnp.float32),
        ],
    )
    def hist(dst_hbm, ones_hbm, zeros_hbm, cnt_hbm, idx_v, ones_v, stage_v,
             cnt_sh):
        c = lax.axis_index("c")
        s = lax.axis_index("s")
        wid = c * NS + s
        pltpu.sync_copy(zeros_hbm, stage_v)
        pltpu.sync_copy(stage_v, cnt_sh.at[pl.ds(s * stripe, stripe)])
        pltpu.sync_copy(ones_hbm, ones_v)
        pltpu.sync_copy(dst_hbm.at[wid], idx_v)
        plsc.subcore_barrier()

        def body(j, carry):
            pltpu.sync_copy(ones_v, cnt_sh.at[idx_v.at[j]], add=True)
            return carry

        lax.fori_loop(0, nchunk, body, 0)
        plsc.subcore_barrier()
        pltpu.sync_copy(cnt_sh.at[pl.ds(s * stripe, stripe)], stage_v)
        pltpu.sync_copy(stage_v, cnt_hbm.at[pl.ds(c * n2 + s * stripe, stripe)])

    return hist


def _scatter_kernel(n, n2, nchunk, stripe, dh):
    half = stripe // 2

    @functools.partial(
        pl.kernel,
        out_type=jax.ShapeDtypeStruct((NC, n2, dh), jnp.float32),
        mesh=_MESH,
        scratch_types=[
            pltpu.VMEM((nchunk, CHUNK), jnp.int32),
            pltpu.VMEM((nchunk, CHUNK), jnp.int32),
            pltpu.VMEM((CHUNK, dh), jnp.float32),
            pltpu.VMEM((half, dh), jnp.float32),
            pltpu.VMEM_SHARED((n2, dh), jnp.float32),
            pltpu.SemaphoreType.DMA,
        ],
        compiler_params=pltpu.CompilerParams(use_tc_tiling_on_sc=False),
    )
    def scat(src_hbm, dst_hbm, g_hbm, zeros_hbm, acc_hbm,
             si_v, di_v, rows_v, stage_v, acc_sh, sem):
        c = lax.axis_index("c")
        s = lax.axis_index("s")
        pltpu.sync_copy(zeros_hbm, stage_v)
        for k in range(2):
            pltpu.sync_copy(
                stage_v, acc_sh.at[pl.ds(s * stripe + k * half, half)])
        pltpu.sync_copy(src_hbm.at[s], si_v)
        pltpu.sync_copy(dst_hbm.at[s], di_v)
        off = (c * n).astype(jnp.int32)

        def add_off(j, carry):
            for k in range(CHUNK // 16):
                sl = pl.ds(k * 16, 16)
                si_v[j, sl] = si_v[j, sl] + off
            return carry

        lax.fori_loop(0, nchunk, add_off, 0)
        plsc.subcore_barrier()

        def body(j, carry):
            pltpu.async_copy(g_hbm.at[si_v.at[j]], rows_v, sem).wait()
            pltpu.sync_copy(rows_v, acc_sh.at[di_v.at[j]], add=True)
            return carry

        lax.fori_loop(0, nchunk, body, 0)
        plsc.subcore_barrier()
        for k in range(2):
            pltpu.sync_copy(
                acc_sh.at[pl.ds(s * stripe + k * half, half)], stage_v)
            pltpu.sync_copy(
                stage_v, acc_hbm.at[c, pl.ds(s * stripe + k * half, half)])

    return scat


def _mm_body(x_ref, w_ref, c0_ref, c1_ref, g_ref):
    deg = c0_ref[...] + c1_ref[...] + 1.0
    dis = lax.rsqrt(deg)
    h = jnp.dot(x_ref[...], w_ref[0], preferred_element_type=jnp.float32)
    g_ref[0] = h * dis


def _fin_body(a0_ref, a1_ref, g0_ref, g1_ref, c0_ref, c1_ref, b_ref, o_ref):
    dis = lax.rsqrt(c0_ref[...] + c1_ref[...] + 1.0)
    o_ref[...] = jnp.concatenate(
        [dis * (a0_ref[...] + g0_ref[...]), dis * (a1_ref[...] + g1_ref[...])],
        axis=1) + b_ref[...]


def kernel(x, edge_index, W, b):
    n, d_in = x.shape
    d_out = W.shape[1]
    dh = d_out // 2
    e = edge_index.shape[1]

    stripe = -(-n // (NS * 32)) * 32
    n2 = stripe * NS
    epw_h = -(-e // (NW * CHUNK)) * CHUNK
    nch_h = epw_h // CHUNK
    e2_h = epw_h * NW
    epw_s = -(-e // (NS * CHUNK)) * CHUNK
    nch_s = epw_s // CHUNK
    e2_s = epw_s * NS

    src = edge_index[0].astype(jnp.int32)
    dst = edge_index[1].astype(jnp.int32)
    dst_h = jnp.concatenate(
        [dst, jnp.full((e2_h - e,), n, jnp.int32)]).reshape(NW, nch_h, CHUNK)
    src_s = jnp.concatenate(
        [src, jnp.zeros((e2_s - e,), jnp.int32)]).reshape(NS, nch_s, CHUNK)
    dst_s = jnp.concatenate(
        [dst, jnp.full((e2_s - e,), n, jnp.int32)]).reshape(NS, nch_s, CHUNK)

    ones_c = jnp.ones((CHUNK,), jnp.float32)
    zeros_s = jnp.zeros((stripe,), jnp.float32)
    zeros_sd = jnp.zeros((stripe // 2, dh), jnp.float32)

    cnt = _hist_kernel(n2, nch_h, stripe)(dst_h, ones_c, zeros_s)
    cnt = cnt.reshape(NC, n2)
    c0 = cnt[0, :n].reshape(n, 1)
    c1 = cnt[1, :n].reshape(n, 1)

    blk = 1000
    grid = n // blk
    g2 = pl.pallas_call(
        _mm_body,
        grid=(NC, grid),
        in_specs=[
            pl.BlockSpec((blk, d_in), lambda j, i: (i, 0)),
            pl.BlockSpec((1, d_in, dh), lambda j, i: (j, 0, 0)),
            pl.BlockSpec((blk, 1), lambda j, i: (i, 0)),
            pl.BlockSpec((blk, 1), lambda j, i: (i, 0)),
        ],
        out_specs=pl.BlockSpec((1, blk, dh), lambda j, i: (j, i, 0)),
        out_shape=jax.ShapeDtypeStruct((NC, n, dh), jnp.float32),
    )(x, jnp.moveaxis(W.reshape(d_in, NC, dh), 1, 0), c0, c1)

    acc = _scatter_kernel(n, n2, nch_s, stripe, dh)(
        src_s, dst_s, g2.reshape(NC * n, dh), zeros_sd)

    out = pl.pallas_call(
        _fin_body,
        grid=(grid,),
        in_specs=[
            pl.BlockSpec((blk, dh), lambda i: (i, 0)),
            pl.BlockSpec((blk, dh), lambda i: (i, 0)),
            pl.BlockSpec((blk, dh), lambda i: (i, 0)),
            pl.BlockSpec((blk, dh), lambda i: (i, 0)),
            pl.BlockSpec((blk, 1), lambda i: (i, 0)),
            pl.BlockSpec((blk, 1), lambda i: (i, 0)),
            pl.BlockSpec((1, d_out), lambda i: (0, 0)),
        ],
        out_specs=pl.BlockSpec((blk, d_out), lambda i: (i, 0)),
        out_shape=jax.ShapeDtypeStruct((n, d_out), jnp.float32),
    )(acc[0, :n], acc[1, :n], g2[0], g2[1], c0, c1, b.reshape(1, d_out))
    return out
